# baseline (device time: 39285 ns/iter reference)
import jax
import jax.numpy as jnp
from jax import lax
from jax.experimental import pallas as pl
from jax.experimental.pallas import tpu as pltpu

N_DEV = 32
ROW = 8
COL = 4
R_HOPS = ROW // 2
L_HOPS = ROW // 2 - 1


def kernel(q, k, v):
    s_per, d = q.shape
    scale = 1.0 / (d ** 0.5)

    def body(q_ref, k_ref, v_ref, out_ref,
             qa, rbuf, lbuf, sblk, rblk, sblk_l, rblk_l,
             qa_sems, brk_send, brk_recv, brv_send, brv_recv,
             blk_send, blk_recv, blv_send, blv_recv,
             cs_sems, cr_sems, cls_sems, clr_sems):
        p = lax.axis_index("i")
        base = (p // ROW) * ROW
        w = p % ROW
        z = p // ROW
        right = base + (w + 1) % ROW
        left = base + (w - 1) % ROW
        pz1 = (z ^ 1) * ROW + w
        pz2 = (z ^ 2) * ROW + w
        pz3 = (z ^ 3) * ROW + w

        barrier_sem = pltpu.get_barrier_semaphore()
        for nbr in [left, right, pz1, pz2, pz3]:
            pl.semaphore_signal(
                barrier_sem, inc=1,
                device_id=(nbr,), device_id_type=pl.DeviceIdType.MESH,
            )
        pl.semaphore_wait(barrier_sem, 5)

        k_bf = k_ref[:, :].astype(jnp.bfloat16)
        v_bf = v_ref[:, :].astype(jnp.bfloat16)
        rbuf[0, :s_per, :] = k_bf
        rbuf[0, s_per:, :] = v_bf
        lbuf[0, :s_per, :] = k_bf
        lbuf[0, s_per:, :] = v_bf
        qa[0, :, :] = (q_ref[:, :] * scale).astype(jnp.bfloat16)

        def make(src, dst, send, recv, dev):
            return pltpu.make_async_remote_copy(
                src_ref=src, dst_ref=dst, send_sem=send, recv_sem=recv,
                device_id=(dev,), device_id_type=pl.DeviceIdType.MESH,
            )

        q1_desc = make(qa.at[0], qa.at[1], qa_sems.at[0], qa_sems.at[1], pz1)
        q2a_desc = make(qa.at[0], qa.at[2], qa_sems.at[2], qa_sems.at[3], pz2)
        q2b_desc = make(qa.at[1], qa.at[3], qa_sems.at[4], qa_sems.at[5], pz2)

        def half_ring(buf, send, recv, dev, hops, lo):
            return [
                make(buf.at[h, pl.ds(lo, s_per), :],
                     buf.at[h + 1, pl.ds(lo, s_per), :],
                     send.at[h + 1], recv.at[h + 1], dev)
                for h in range(hops)
            ]

        rk_desc = half_ring(rbuf, brk_send, brk_recv, right, R_HOPS, 0)
        rv_desc = half_ring(rbuf, brv_send, brv_recv, right, R_HOPS, s_per)
        lk_desc = half_ring(lbuf, blk_send, blk_recv, left, L_HOPS, 0)
        lv_desc = half_ring(lbuf, blv_send, blv_recv, left, L_HOPS, s_per)
        c_desc = [
            make(sblk.at[s - 1], rblk.at[s - 1],
                 cs_sems.at[s - 1], cr_sems.at[s - 1],
                 (z ^ s) * ROW + w)
            for s in (1, 2, 3)
        ]
        cl_desc = [
            make(sblk_l.at[s - 1], rblk_l.at[s - 1],
                 cls_sems.at[s - 1], clr_sems.at[s - 1],
                 (z ^ s) * ROW + w)
            for s in (1, 2, 3)
        ]

        rk_desc[0].start()
        rv_desc[0].start()
        lk_desc[0].start()
        lv_desc[0].start()
        q1_desc.start()

        ones = jnp.ones((s_per, d), dtype=jnp.bfloat16)

        def scores_pr(q_blk, buf, slot):
            scores = jax.lax.dot_general(
                q_blk, buf[slot, :s_per, :],
                (((1,), (1,)), ((), ())),
                preferred_element_type=jnp.float32,
            )
            return jnp.exp(scores).astype(jnp.bfloat16)

        def pv(pr, buf, slot):
            v_aug = jnp.concatenate([buf[slot, s_per:, :], ones], axis=1)
            return jax.lax.dot_general(
                pr, v_aug,
                (((1,), (0,)), ((), ())),
                preferred_element_type=jnp.float32,
            )

        def fold(q_blk, buf, slot):
            return pv(scores_pr(q_blk, buf, slot), buf, slot)

        q1_desc.wait_recv()
        q2a_desc.start()
        q2b_desc.start()
        q01 = jnp.concatenate([qa[0, :, :], qa[1, :, :]], axis=0)
        acc01 = fold(q01, rbuf, 0)

        rk_desc[0].wait_recv()
        rk_desc[1].start()
        pr_r = scores_pr(q01, rbuf, 1)
        rv_desc[0].wait_recv()
        rv_desc[1].start()
        acc01 = acc01 + pv(pr_r, rbuf, 1)
        lk_desc[0].wait_recv()
        lk_desc[1].start()
        pr_l = scores_pr(q01, lbuf, 1)
        lv_desc[0].wait_recv()
        lv_desc[1].start()
        acc01 = acc01 + pv(pr_l, lbuf, 1)

        q2a_desc.wait_recv()
        q2b_desc.wait_recv()
        q23 = jnp.concatenate([qa[2, :, :], qa[3, :, :]], axis=0)
        acc23 = fold(q23, rbuf, 0) + fold(q23, rbuf, 1) + fold(q23, lbuf, 1)

        for h in range(2, R_HOPS):
            rk_desc[h - 1].wait_recv()
            rk_desc[h].start()
            pr_r01 = scores_pr(q01, rbuf, h)
            pr_r23 = scores_pr(q23, rbuf, h)
            rv_desc[h - 1].wait_recv()
            rv_desc[h].start()
            acc01 = acc01 + pv(pr_r01, rbuf, h)
            acc23 = acc23 + pv(pr_r23, rbuf, h)
            lk_desc[h - 1].wait_recv()
            if h < L_HOPS:
                lk_desc[h].start()
            pr_l01 = scores_pr(q01, lbuf, h)
            pr_l23 = scores_pr(q23, lbuf, h)
            lv_desc[h - 1].wait_recv()
            if h < L_HOPS:
                lv_desc[h].start()
            acc01 = acc01 + pv(pr_l01, lbuf, h)
            acc23 = acc23 + pv(pr_l23, lbuf, h)

        def pack(blk):
            return (
                blk[:, :d].astype(jnp.bfloat16),
                jnp.reshape(blk[:, d], (s_per // d, d)),
            )

        rk_desc[R_HOPS - 1].wait_recv()
        pr_t23 = scores_pr(q23, rbuf, R_HOPS)
        rv_desc[R_HOPS - 1].wait_recv()
        acc23 = acc23 + pv(pr_t23, rbuf, R_HOPS)
        sblk[1, :, :], sblk_l[1, :, :] = pack(acc23[:s_per, :])
        sblk[2, :, :], sblk_l[2, :, :] = pack(acc23[s_per:, :])
        c_desc[1].start()
        c_desc[2].start()
        cl_desc[1].start()
        cl_desc[2].start()
        acc01 = acc01 + fold(q01, rbuf, R_HOPS)
        sblk[0, :, :], sblk_l[0, :, :] = pack(acc01[s_per:, :])
        c_desc[0].start()
        cl_desc[0].start()
        for desc in c_desc + cl_desc:
            desc.wait_recv()
        aug_v = (acc01[:s_per, :d]
                 + rblk[0, :, :].astype(jnp.float32)
                 + rblk[1, :, :].astype(jnp.float32)
                 + rblk[2, :, :].astype(jnp.float32))
        l_pack = (jnp.reshape(acc01[:s_per, d], (s_per // d, d))
                  + rblk_l[0, :, :] + rblk_l[1, :, :] + rblk_l[2, :, :])
        out_ref[:, :] = aug_v / jnp.reshape(l_pack, (s_per, 1))

        for desc in [q1_desc, q2a_desc, q2b_desc] + c_desc + cl_desc \
                + rk_desc + rv_desc + lk_desc + lv_desc:
            desc.wait_send()

    return pl.pallas_call(
        body,
        out_shape=jax.ShapeDtypeStruct((s_per, d), jnp.float32),
        in_specs=[
            pl.BlockSpec(memory_space=pltpu.VMEM),
            pl.BlockSpec(memory_space=pltpu.VMEM),
            pl.BlockSpec(memory_space=pltpu.VMEM),
        ],
        out_specs=pl.BlockSpec(memory_space=pltpu.VMEM),
        scratch_shapes=[
            pltpu.VMEM((COL, s_per, d), jnp.bfloat16),
            pltpu.VMEM((R_HOPS + 1, 2 * s_per, d), jnp.bfloat16),
            pltpu.VMEM((L_HOPS + 1, 2 * s_per, d), jnp.bfloat16),
            pltpu.VMEM((3, s_per, d), jnp.bfloat16),
            pltpu.VMEM((3, s_per, d), jnp.bfloat16),
            pltpu.VMEM((3, s_per // d, d), jnp.float32),
            pltpu.VMEM((3, s_per // d, d), jnp.float32),
            pltpu.SemaphoreType.DMA((6,)),
            pltpu.SemaphoreType.DMA((R_HOPS + 1,)),
            pltpu.SemaphoreType.DMA((R_HOPS + 1,)),
            pltpu.SemaphoreType.DMA((R_HOPS + 1,)),
            pltpu.SemaphoreType.DMA((R_HOPS + 1,)),
            pltpu.SemaphoreType.DMA((L_HOPS + 1,)),
            pltpu.SemaphoreType.DMA((L_HOPS + 1,)),
            pltpu.SemaphoreType.DMA((L_HOPS + 1,)),
            pltpu.SemaphoreType.DMA((L_HOPS + 1,)),
            pltpu.SemaphoreType.DMA((3,)),
            pltpu.SemaphoreType.DMA((3,)),
            pltpu.SemaphoreType.DMA((3,)),
            pltpu.SemaphoreType.DMA((3,)),
        ],
        compiler_params=pltpu.CompilerParams(collective_id=0),
    )(q, k, v)
